# baseline (speedup 1.0000x reference)
import functools

import jax
import jax.numpy as jnp
from jax import lax
from jax.experimental import pallas as pl
from jax.experimental.pallas import tpu as pltpu
from jax.experimental.pallas import tpu_sc as plsc

HDIM = 64
LANES = 16
NC = 2
NS = 16
NW = NC * NS
CH = 128
GROUPS = CH // LANES
NBUF = 2
EPS = 1e-5


def _rsqrt(x):
    i = plsc.bitcast(x, jnp.int32)
    i = jnp.int32(0x5F3759DF) - lax.shift_right_logical(i, 1)
    y = plsc.bitcast(i, jnp.float32)
    for _ in range(3):
        y = y * (1.5 - 0.5 * x * y * y)
    return y


@functools.lru_cache(maxsize=None)
def _build(nch, n_batch):
    mesh = plsc.VectorSubcoreMesh(core_axis_name="c", subcore_axis_name="s")

    @functools.partial(
        pl.kernel,
        mesh=mesh,
        compiler_params=pltpu.CompilerParams(needs_layout_passes=False),
        out_type=jax.ShapeDtypeStruct((nch, HDIM, n_batch), jnp.float32),
        scratch_types=[
            pltpu.VMEM((nch, CH), jnp.int32),
            pltpu.VMEM((nch, CH), jnp.int32),
            pltpu.VMEM((NBUF * CH, 2 * HDIM), jnp.float32),
            pltpu.VMEM((NBUF * HDIM, CH), jnp.float32),
            pltpu.VMEM((NBUF * 2, CH), jnp.float32),
            pltpu.VMEM((HDIM * LANES,), jnp.float32),
            pltpu.VMEM((HDIM * LANES,), jnp.float32),
            pltpu.SemaphoreType.DMA((NBUF,)),
            pltpu.SemaphoreType.DMA((NBUF,)),
        ],
    )
    def kern(idxh_hbm, idxp_hbm, table_hbm, gexp_hbm, bexp_hbm, out_hbm,
             idxh_v, idxp_v, rows_v, obuf_v, mst_v, gexp_v, bexp_v,
             gsem, osem):
        wid = lax.axis_index("s") * NC + lax.axis_index("c")
        pltpu.sync_copy(idxh_hbm.at[wid], idxh_v)
        pltpu.sync_copy(idxp_hbm.at[wid], idxp_v)
        pltpu.sync_copy(gexp_hbm, gexp_v)
        pltpu.sync_copy(bexp_hbm, bexp_v)
        rid0 = lax.iota(jnp.int32, LANES)

        def g_copy(ci, b):
            return pltpu.make_async_copy(
                table_hbm.at[idxh_v.at[ci]],
                rows_v.at[pl.ds(b * CH, CH)], gsem.at[b])

        def o_copy(ci, b):
            return pltpu.make_async_copy(
                obuf_v.at[pl.ds(b * HDIM, HDIM)],
                out_hbm.at[ci, :, pl.ds(wid * CH, CH)], osem.at[b])

        def compute(ci, b):
            rowbase = b * CH
            obase = b * HDIM

            @plsc.parallel_loop(0, GROUPS)
            def phase_a(g):
                r = rid0 + (g * LANES + rowbase)
                pv = idxp_v[ci, pl.ds(g * LANES, LANES)]
                acc = jnp.zeros((LANES,), jnp.float32)
                acc2 = jnp.zeros((LANES,), jnp.float32)
                for d in range(HDIM):
                    dcol = ((rid0 + d) & (HDIM - 1)) + pv
                    v = plsc.load_gather(rows_v, [r, dcol])
                    acc = acc + v
                    acc2 = acc2 + v * v
                mean = acc * (1.0 / HDIM)
                var = acc2 * (1.0 / HDIM) - mean * mean
                mst_v[2 * b, pl.ds(g * LANES, LANES)] = mean
                mst_v[2 * b + 1, pl.ds(g * LANES, LANES)] = _rsqrt(var + EPS)

            means = [mst_v[2 * b, pl.ds(g * LANES, LANES)] for g in range(GROUPS)]
            rstds = [mst_v[2 * b + 1, pl.ds(g * LANES, LANES)] for g in range(GROUPS)]
            rids = [rid0 + (g * LANES + rowbase) for g in range(GROUPS)]
            pvs = [idxp_v[ci, pl.ds(g * LANES, LANES)] for g in range(GROUPS)]
            tids = [rid0 + g * LANES for g in range(GROUPS)]

            @plsc.parallel_loop(0, HDIM, unroll=2)
            def phase_b(d):
                gd = gexp_v[pl.ds(d * LANES, LANES)]
                bd = bexp_v[pl.ds(d * LANES, LANES)]
                dcol = (rid0 + d) & (HDIM - 1)
                for g in range(GROUPS):
                    v = plsc.load_gather(rows_v, [rids[g], dcol + pvs[g]])
                    o = (v - means[g]) * rstds[g] * gd + bd
                    plsc.store_scatter(obuf_v, [dcol + obase, tids[g]], o)

        for b in range(NBUF):
            g_copy(b, b).start()

        def steady(i, _):
            i0 = i * NBUF
            for b in range(NBUF):
                ci = i0 + b
                g_copy(ci, b).wait()

                @pl.when(ci >= NBUF)
                def _():
                    o_copy(ci - NBUF, b).wait()

                compute(ci, b)
                o_copy(ci, b).start()

                @pl.when(ci + NBUF < nch)
                def _():
                    g_copy(ci + NBUF, b).start()
            return 0

        lax.fori_loop(0, nch // NBUF, steady, 0)
        for b in range(NBUF):
            o_copy(nch - NBUF + b, b).wait()

    return kern


def kernel(input, table, gamma, beta):
    B, L = input.shape
    V, H = table.shape
    idx4 = input.reshape(NW, CH, L).transpose(0, 2, 1).astype(jnp.int32)
    idxh = idx4 >> 1
    idxp = (idx4 & 1) << 6
    table2 = table.reshape(V // 2, 2 * H)
    diag = (jnp.arange(H)[:, None] + jnp.arange(LANES)[None, :]) % H
    gexp = gamma.astype(jnp.float32)[diag].reshape(H * LANES)
    bexp = beta.astype(jnp.float32)[diag].reshape(H * LANES)
    o2 = _build(L, B)(idxh, idxp, table2, gexp, bexp)
    return jnp.transpose(o2, (2, 0, 1))


# speedup vs baseline: 1.3984x; 1.3984x over previous
import functools

import jax
import jax.numpy as jnp
from jax import lax
from jax.experimental import pallas as pl
from jax.experimental.pallas import tpu as pltpu
from jax.experimental.pallas import tpu_sc as plsc

HDIM = 64
LANES = 16
NC = 2
NS = 16
NW = NC * NS
CH = 128
GROUPS = CH // LANES
NBUF = 2
EPS = 1e-5


def _rsqrt(x):
    i = plsc.bitcast(x, jnp.int32)
    i = jnp.int32(0x5F3759DF) - lax.shift_right_logical(i, 1)
    y = plsc.bitcast(i, jnp.float32)
    for _ in range(3):
        y = y * (1.5 - 0.5 * x * y * y)
    return y


@functools.lru_cache(maxsize=None)
def _build(nch, n_batch):
    mesh = plsc.VectorSubcoreMesh(core_axis_name="c", subcore_axis_name="s")

    @functools.partial(
        pl.kernel,
        mesh=mesh,
        compiler_params=pltpu.CompilerParams(needs_layout_passes=False),
        out_type=jax.ShapeDtypeStruct((nch, HDIM, n_batch), jnp.float32),
        scratch_types=[
            pltpu.VMEM((nch, CH), jnp.int32),
            pltpu.VMEM((nch, CH), jnp.int32),
            pltpu.VMEM((NBUF * CH, 2 * HDIM), jnp.float32),
            pltpu.VMEM((NBUF * HDIM, CH), jnp.float32),
            pltpu.VMEM((NBUF * 2, CH), jnp.float32),
            pltpu.VMEM((HDIM * LANES,), jnp.float32),
            pltpu.VMEM((HDIM * LANES,), jnp.float32),
            pltpu.SemaphoreType.DMA((NBUF,)),
            pltpu.SemaphoreType.DMA((NBUF,)),
        ],
    )
    def kern(idxh_hbm, idxp_hbm, table_hbm, gexp_hbm, bexp_hbm, out_hbm,
             idxh_v, idxp_v, rows_v, obuf_v, mst_v, gexp_v, bexp_v,
             gsem, osem):
        wid = lax.axis_index("s") * NC + lax.axis_index("c")
        pltpu.sync_copy(idxh_hbm.at[wid], idxh_v)
        pltpu.sync_copy(idxp_hbm.at[wid], idxp_v)
        pltpu.sync_copy(gexp_hbm, gexp_v)
        pltpu.sync_copy(bexp_hbm, bexp_v)
        rid0 = lax.iota(jnp.int32, LANES)

        def g_copy(ci, b):
            return pltpu.make_async_copy(
                table_hbm.at[idxh_v.at[ci]],
                rows_v.at[pl.ds(b * CH, CH)], gsem.at[b])

        def o_copy(ci, b):
            return pltpu.make_async_copy(
                obuf_v.at[pl.ds(b * HDIM, HDIM)],
                out_hbm.at[ci, :, pl.ds(wid * CH, CH)], osem.at[b])

        def compute(ci, b):
            rowbase = b * CH
            obase = b * HDIM
            pvs = [idxp_v[ci, pl.ds(g * LANES, LANES)] for g in range(GROUPS)]
            rids = [rid0 + (g * LANES + rowbase) for g in range(GROUPS)]
            tids = [rid0 + g * LANES for g in range(GROUPS)]

            zero = jnp.zeros((LANES,), jnp.float32)
            carry0 = tuple([zero] * (2 * GROUPS))

            def stats_body(d, carry):
                accs = list(carry)
                c63 = (rid0 + d) & (HDIM - 1)
                for g in range(GROUPS):
                    v = plsc.load_gather(rows_v, [rids[g], c63 + pvs[g]])
                    accs[g] = accs[g] + v
                    accs[GROUPS + g] = accs[GROUPS + g] + v * v
                return tuple(accs)

            res = plsc.parallel_loop(0, HDIM, carry=carry0)(stats_body)
            means, rstds = [], []
            for g in range(GROUPS):
                mean = res[g] * (1.0 / HDIM)
                var = res[GROUPS + g] * (1.0 / HDIM) - mean * mean
                means.append(mean)
                rstds.append(_rsqrt(var + EPS))

            @plsc.parallel_loop(0, HDIM, unroll=2)
            def phase_b(d):
                gd = gexp_v[pl.ds(d * LANES, LANES)]
                bd = bexp_v[pl.ds(d * LANES, LANES)]
                c63 = (rid0 + d) & (HDIM - 1)
                crow = c63 + obase
                for g in range(GROUPS):
                    v = plsc.load_gather(rows_v, [rids[g], c63 + pvs[g]])
                    o = (v - means[g]) * rstds[g] * gd + bd
                    plsc.store_scatter(obuf_v, [crow, tids[g]], o)

        for b in range(NBUF):
            g_copy(b, b).start()

        def steady(i, _):
            i0 = i * NBUF
            for b in range(NBUF):
                ci = i0 + b
                g_copy(ci, b).wait()

                @pl.when(ci >= NBUF)
                def _():
                    o_copy(ci - NBUF, b).wait()

                compute(ci, b)
                o_copy(ci, b).start()

                @pl.when(ci + NBUF < nch)
                def _():
                    g_copy(ci + NBUF, b).start()
            return 0

        lax.fori_loop(0, nch // NBUF, steady, 0)
        for b in range(NBUF):
            o_copy(nch - NBUF + b, b).wait()

    return kern


def kernel(input, table, gamma, beta):
    B, L = input.shape
    V, H = table.shape
    idx4 = input.reshape(NW, CH, L).transpose(0, 2, 1).astype(jnp.int32)
    idxh = idx4 >> 1
    idxp = (idx4 & 1) << 6
    table2 = table.reshape(V // 2, 2 * H)
    diag = (jnp.arange(H)[:, None] + jnp.arange(LANES)[None, :]) % H
    gexp = gamma.astype(jnp.float32)[diag].reshape(H * LANES)
    bexp = beta.astype(jnp.float32)[diag].reshape(H * LANES)
    o2 = _build(L, B)(idxh, idxp, table2, gexp, bexp)
    return jnp.transpose(o2, (2, 0, 1))


# speedup vs baseline: 1.4883x; 1.0643x over previous
import functools

import jax
import jax.numpy as jnp
from jax import lax
from jax.experimental import pallas as pl
from jax.experimental.pallas import tpu as pltpu
from jax.experimental.pallas import tpu_sc as plsc

HDIM = 64
LANES = 16
NC = 2
NS = 16
NW = NC * NS
CH = 128
GROUPS = CH // LANES
NBUF = 2
EPS = 1e-5


def _rsqrt(x):
    i = plsc.bitcast(x, jnp.int32)
    i = jnp.int32(0x5F3759DF) - lax.shift_right_logical(i, 1)
    y = plsc.bitcast(i, jnp.float32)
    for _ in range(3):
        y = y * (1.5 - 0.5 * x * y * y)
    return y


@functools.lru_cache(maxsize=None)
def _build(nch, n_batch):
    mesh = plsc.VectorSubcoreMesh(core_axis_name="c", subcore_axis_name="s")

    @functools.partial(
        pl.kernel,
        mesh=mesh,
        compiler_params=pltpu.CompilerParams(needs_layout_passes=False),
        out_type=jax.ShapeDtypeStruct((nch, HDIM, n_batch), jnp.float32),
        scratch_types=[
            pltpu.VMEM((nch, CH), jnp.int32),
            pltpu.VMEM((nch, CH), jnp.int32),
            pltpu.VMEM((NBUF * CH, 2 * HDIM), jnp.float32),
            pltpu.VMEM((NBUF * HDIM, CH), jnp.float32),
            pltpu.VMEM((NBUF * 2, CH), jnp.float32),
            pltpu.VMEM((HDIM * LANES,), jnp.float32),
            pltpu.VMEM((HDIM * LANES,), jnp.float32),
            pltpu.SemaphoreType.DMA((NBUF,)),
            pltpu.SemaphoreType.DMA((NBUF,)),
        ],
    )
    def kern(idxh_hbm, idxp_hbm, table_hbm, gexp_hbm, bexp_hbm, out_hbm,
             idxh_v, idxp_v, rows_v, obuf_v, mst_v, gexp_v, bexp_v,
             gsem, osem):
        wid = lax.axis_index("s") * NC + lax.axis_index("c")
        pltpu.sync_copy(idxh_hbm.at[wid], idxh_v)
        pltpu.sync_copy(idxp_hbm.at[wid], idxp_v)
        pltpu.sync_copy(gexp_hbm, gexp_v)
        pltpu.sync_copy(bexp_hbm, bexp_v)
        rid0 = lax.iota(jnp.int32, LANES)

        def g_copy(ci, b):
            return pltpu.make_async_copy(
                table_hbm.at[idxh_v.at[ci]],
                rows_v.at[pl.ds(b * CH, CH)], gsem.at[b])

        def o_copy(ci, b):
            return pltpu.make_async_copy(
                obuf_v.at[pl.ds(b * HDIM, HDIM)],
                out_hbm.at[ci, :, pl.ds(wid * CH, CH)], osem.at[b])

        def compute(ci, b):
            rowbase = b * CH
            obase = b * HDIM
            pvs = [idxp_v[ci, pl.ds(g * LANES, LANES)] for g in range(GROUPS)]
            rids = [rid0 + (g * LANES + rowbase) for g in range(GROUPS)]
            tids = [rid0 + g * LANES for g in range(GROUPS)]

            zero = jnp.zeros((LANES,), jnp.float32)
            carry0 = tuple([zero] * (2 * GROUPS))

            def stats_body(d, carry):
                accs = list(carry)
                c63 = (rid0 + d) & (HDIM - 1)
                for g in range(GROUPS):
                    v = plsc.load_gather(rows_v, [rids[g], c63 + pvs[g]])
                    accs[g] = accs[g] + v
                    accs[GROUPS + g] = accs[GROUPS + g] + v * v
                return tuple(accs)

            res = plsc.parallel_loop(0, HDIM, carry=carry0)(stats_body)
            means, rstds = [], []
            for g in range(GROUPS):
                mean = res[g] * (1.0 / HDIM)
                var = res[GROUPS + g] * (1.0 / HDIM) - mean * mean
                means.append(mean)
                rstds.append(_rsqrt(var + EPS))

            @plsc.parallel_loop(0, HDIM, unroll=2)
            def phase_b(d):
                gd = gexp_v[pl.ds(d * LANES, LANES)]
                bd = bexp_v[pl.ds(d * LANES, LANES)]
                c63 = (rid0 + d) & (HDIM - 1)
                crow = c63 + obase
                for g in range(GROUPS):
                    v = plsc.load_gather(rows_v, [rids[g], c63 + pvs[g]])
                    o = (v - means[g]) * rstds[g] * gd + bd
                    plsc.store_scatter(obuf_v, [crow, tids[g]], o)

        for b in range(NBUF):
            g_copy(b, b).start()

        def steady(i, _):
            i0 = i * NBUF
            for b in range(NBUF):
                ci = i0 + b
                g_copy(ci, b).wait()

                @pl.when(ci >= NBUF)
                def _():
                    o_copy(ci - NBUF, b).wait()

                compute(ci, b)
                o_copy(ci, b).start()

                @pl.when(ci + NBUF < nch)
                def _():
                    g_copy(ci + NBUF, b).start()
            return 0

        lax.fori_loop(0, nch // NBUF, steady, 0)
        for b in range(NBUF):
            o_copy(nch - NBUF + b, b).wait()

    return kern


def kernel(input, table, gamma, beta):
    B, L = input.shape
    V, H = table.shape
    idx4 = input.reshape(NW, CH, L).transpose(0, 2, 1).astype(jnp.int32)
    idxh = idx4
    idxp = jnp.zeros_like(idx4)
    table2 = jnp.pad(table, ((0, 0), (0, H)))
    diag = (jnp.arange(H)[:, None] + jnp.arange(LANES)[None, :]) % H
    gexp = gamma.astype(jnp.float32)[diag].reshape(H * LANES)
    bexp = beta.astype(jnp.float32)[diag].reshape(H * LANES)
    o2 = _build(L, B)(idxh, idxp, table2, gexp, bexp)
    return jnp.transpose(o2, (2, 0, 1))


# speedup vs baseline: 2.2856x; 1.5358x over previous
import functools

import jax
import jax.numpy as jnp
from jax import lax
from jax.experimental import pallas as pl
from jax.experimental.pallas import tpu as pltpu
from jax.experimental.pallas import tpu_sc as plsc

HDIM = 64
LANES = 16
NC = 2
NS = 16
NW = NC * NS
CH = 128
GROUPS = CH // LANES
NBUF = 2
EPS = 1e-5


def _rsqrt(x):
    i = plsc.bitcast(x, jnp.int32)
    i = jnp.int32(0x5F3759DF) - lax.shift_right_logical(i, 1)
    y = plsc.bitcast(i, jnp.float32)
    for _ in range(3):
        y = y * (1.5 - 0.5 * x * y * y)
    return y


@functools.lru_cache(maxsize=None)
def _build_fmt(V):
    nblk = (V + CH - 1) // CH
    nfull = V // CH
    rem = V - nfull * CH
    vpad = nblk * CH
    mesh = plsc.VectorSubcoreMesh(core_axis_name="c", subcore_axis_name="s")

    @functools.partial(
        pl.kernel,
        mesh=mesh,
        compiler_params=pltpu.CompilerParams(needs_layout_passes=False),
        out_type=jax.ShapeDtypeStruct((vpad, 2 * HDIM), jnp.float32),
        scratch_types=[
            pltpu.VMEM((NBUF * HDIM, CH), jnp.float32),
            pltpu.VMEM((NBUF * CH, 2 * HDIM), jnp.float32),
            pltpu.SemaphoreType.DMA((NBUF,)),
            pltpu.SemaphoreType.DMA((NBUF,)),
        ],
    )
    def kern(tt_hbm, tail_hbm, out_hbm, in_v, ob_v, isem, osem):
        wid = lax.axis_index("s") * NC + lax.axis_index("c")
        rid0 = lax.iota(jnp.int32, LANES)
        nper = (nfull + NW - 1) // NW

        def blk_of(k, b):
            del b
            return k * NW + wid

        def i_copy(blk, b, w):
            return pltpu.make_async_copy(
                tt_hbm.at[:, pl.ds(blk * CH, w)],
                in_v.at[pl.ds(b * HDIM, HDIM), pl.ds(0, w)], isem.at[b])

        def o_copy(blk, b, w):
            return pltpu.make_async_copy(
                ob_v.at[pl.ds(b * CH, w)],
                out_hbm.at[pl.ds(blk * CH, w)], osem.at[b])

        def transpose_block(b, width_jb):
            ibase = b * HDIM
            obase = b * CH

            @plsc.parallel_loop(0, HDIM)
            def tr(d0):
                c63 = (rid0 + d0) & (HDIM - 1)
                src_r = c63 + ibase
                for jb in range(width_jb):
                    jv = rid0 + jb * LANES
                    v = plsc.load_gather(in_v, [src_r, jv])
                    plsc.store_scatter(ob_v, [jv + obase, c63], v)

        for b in range(NBUF):
            @pl.when(blk_of(b, b) < nfull)
            def _():
                i_copy(blk_of(b, b), b, CH).start()

        def steady(k, _):
            for b in range(NBUF):
                ki = k * NBUF + b
                blk = blk_of(ki, b)

                @pl.when(blk < nfull)
                def _():
                    i_copy(blk, b, CH).wait()

                    @pl.when(ki >= NBUF)
                    def _():
                        o_copy(blk_of(ki - NBUF, b), b, CH).wait()

                    transpose_block(b, GROUPS)
                    o_copy(blk, b, CH).start()
                    nxt = blk_of(ki + NBUF, b)

                    @pl.when(nxt < nfull)
                    def _():
                        i_copy(nxt, b, CH).start()
            return 0

        lax.fori_loop(0, (nper + NBUF - 1) // NBUF, steady, 0)
        for b in range(NBUF):
            o_copy(0, b, CH).wait()

        if rem:
            @pl.when(wid == 0)
            def _():
                pltpu.sync_copy(tail_hbm, ob_v.at[pl.ds(0, rem)])
                pltpu.sync_copy(ob_v.at[pl.ds(0, rem)],
                                out_hbm.at[pl.ds(nfull * CH, rem)])

    return kern


@functools.lru_cache(maxsize=None)
def _build(nch, n_batch, vpad):
    mesh = plsc.VectorSubcoreMesh(core_axis_name="c", subcore_axis_name="s")

    @functools.partial(
        pl.kernel,
        mesh=mesh,
        compiler_params=pltpu.CompilerParams(needs_layout_passes=False),
        out_type=jax.ShapeDtypeStruct((nch, HDIM, n_batch), jnp.float32),
        scratch_types=[
            pltpu.VMEM((nch, CH), jnp.int32),
            pltpu.VMEM((NBUF * CH, 2 * HDIM), jnp.float32),
            pltpu.VMEM((NBUF * HDIM, CH), jnp.float32),
            pltpu.VMEM((HDIM * LANES,), jnp.float32),
            pltpu.VMEM((HDIM * LANES,), jnp.float32),
            pltpu.SemaphoreType.DMA((NBUF,)),
            pltpu.SemaphoreType.DMA((NBUF,)),
        ],
    )
    def kern(idx_hbm, table_hbm, gexp_hbm, bexp_hbm, out_hbm,
             idx_v, rows_v, obuf_v, gexp_v, bexp_v, gsem, osem):
        wid = lax.axis_index("s") * NC + lax.axis_index("c")
        pltpu.sync_copy(idx_hbm.at[wid], idx_v)
        pltpu.sync_copy(gexp_hbm, gexp_v)
        pltpu.sync_copy(bexp_hbm, bexp_v)
        rid0 = lax.iota(jnp.int32, LANES)

        def g_copy(ci, b):
            return pltpu.make_async_copy(
                table_hbm.at[idx_v.at[ci]],
                rows_v.at[pl.ds(b * CH, CH)], gsem.at[b])

        def o_copy(ci, b):
            return pltpu.make_async_copy(
                obuf_v.at[pl.ds(b * HDIM, HDIM)],
                out_hbm.at[ci, :, pl.ds(wid * CH, CH)], osem.at[b])

        def compute(ci, b):
            rowbase = b * CH
            obase = b * HDIM
            rids = [rid0 + (g * LANES + rowbase) for g in range(GROUPS)]
            tids = [rid0 + g * LANES for g in range(GROUPS)]

            zero = jnp.zeros((LANES,), jnp.float32)
            carry0 = tuple([zero] * (2 * GROUPS))

            def stats_body(d, carry):
                accs = list(carry)
                c63 = (rid0 + d) & (HDIM - 1)
                for g in range(GROUPS):
                    v = plsc.load_gather(rows_v, [rids[g], c63])
                    accs[g] = accs[g] + v
                    accs[GROUPS + g] = accs[GROUPS + g] + v * v
                return tuple(accs)

            res = plsc.parallel_loop(0, HDIM, carry=carry0)(stats_body)
            means, rstds = [], []
            for g in range(GROUPS):
                mean = res[g] * (1.0 / HDIM)
                var = res[GROUPS + g] * (1.0 / HDIM) - mean * mean
                means.append(mean)
                rstds.append(_rsqrt(var + EPS))

            @plsc.parallel_loop(0, HDIM, unroll=2)
            def phase_b(d):
                gd = gexp_v[pl.ds(d * LANES, LANES)]
                bd = bexp_v[pl.ds(d * LANES, LANES)]
                c63 = (rid0 + d) & (HDIM - 1)
                crow = c63 + obase
                for g in range(GROUPS):
                    v = plsc.load_gather(rows_v, [rids[g], c63])
                    o = (v - means[g]) * rstds[g] * gd + bd
                    plsc.store_scatter(obuf_v, [crow, tids[g]], o)

        for b in range(NBUF):
            g_copy(b, b).start()

        def steady(i, _):
            i0 = i * NBUF
            for b in range(NBUF):
                ci = i0 + b
                g_copy(ci, b).wait()

                @pl.when(ci >= NBUF)
                def _():
                    o_copy(ci - NBUF, b).wait()

                compute(ci, b)
                o_copy(ci, b).start()

                @pl.when(ci + NBUF < nch)
                def _():
                    g_copy(ci + NBUF, b).start()
            return 0

        lax.fori_loop(0, nch // NBUF, steady, 0)
        for b in range(NBUF):
            o_copy(nch - NBUF + b, b).wait()

    return kern


def kernel(input, table, gamma, beta):
    B, L = input.shape
    V, H = table.shape
    idx4 = input.reshape(NW, CH, L).transpose(0, 2, 1).astype(jnp.int32)
    nfull = V // CH
    tail = jnp.pad(table[nfull * CH:], ((0, 0), (0, H)))
    table2 = _build_fmt(V)(table.T, tail)
    diag = (jnp.arange(H)[:, None] + jnp.arange(LANES)[None, :]) % H
    gexp = gamma.astype(jnp.float32)[diag].reshape(H * LANES)
    bexp = beta.astype(jnp.float32)[diag].reshape(H * LANES)
    o2 = _build(L, B, table2.shape[0])(idx4, table2, gexp, bexp)
    return jnp.transpose(o2, (2, 0, 1))


# speedup vs baseline: 2.3055x; 1.0087x over previous
import functools

import jax
import jax.numpy as jnp
from jax import lax
from jax.experimental import pallas as pl
from jax.experimental.pallas import tpu as pltpu
from jax.experimental.pallas import tpu_sc as plsc

HDIM = 64
LANES = 16
NC = 2
NS = 16
NW = NC * NS
CH = 128
GROUPS = CH // LANES
NBUF = 2
EPS = 1e-5


def _rsqrt(x):
    i = plsc.bitcast(x, jnp.int32)
    i = jnp.int32(0x5F3759DF) - lax.shift_right_logical(i, 1)
    y = plsc.bitcast(i, jnp.float32)
    for _ in range(3):
        y = y * (1.5 - 0.5 * x * y * y)
    return y


@functools.lru_cache(maxsize=None)
def _build_fmt(V):
    nblk = (V + CH - 1) // CH
    nfull = V // CH
    rem = V - nfull * CH
    vpad = nblk * CH
    mesh = plsc.VectorSubcoreMesh(core_axis_name="c", subcore_axis_name="s")

    @functools.partial(
        pl.kernel,
        mesh=mesh,
        compiler_params=pltpu.CompilerParams(needs_layout_passes=False),
        out_type=jax.ShapeDtypeStruct((vpad, 2 * HDIM), jnp.float32),
        scratch_types=[
            pltpu.VMEM((NBUF * HDIM, CH), jnp.float32),
            pltpu.VMEM((NBUF * CH, 2 * HDIM), jnp.float32),
            pltpu.SemaphoreType.DMA((NBUF,)),
            pltpu.SemaphoreType.DMA((NBUF,)),
        ],
    )
    def kern(tt_hbm, tail_hbm, out_hbm, in_v, ob_v, isem, osem):
        wid = lax.axis_index("s") * NC + lax.axis_index("c")
        rid0 = lax.iota(jnp.int32, LANES)
        nper = (nfull + NW - 1) // NW

        def blk_of(k, b):
            del b
            return k * NW + wid

        def i_copy(blk, b, w):
            return pltpu.make_async_copy(
                tt_hbm.at[:, pl.ds(blk * CH, w)],
                in_v.at[pl.ds(b * HDIM, HDIM), pl.ds(0, w)], isem.at[b])

        def o_copy(blk, b, w):
            return pltpu.make_async_copy(
                ob_v.at[pl.ds(b * CH, w)],
                out_hbm.at[pl.ds(blk * CH, w)], osem.at[b])

        def transpose_block(b, width_jb):
            ibase = b * HDIM
            obase = b * CH

            @plsc.parallel_loop(0, HDIM)
            def tr(d0):
                c63 = (rid0 + d0) & (HDIM - 1)
                src_r = c63 + ibase
                for jb in range(width_jb):
                    jv = rid0 + jb * LANES
                    v = plsc.load_gather(in_v, [src_r, jv])
                    plsc.store_scatter(ob_v, [jv + obase, c63], v)

        for b in range(NBUF):
            @pl.when(blk_of(b, b) < nfull)
            def _():
                i_copy(blk_of(b, b), b, CH).start()

        def steady(k, _):
            for b in range(NBUF):
                ki = k * NBUF + b
                blk = blk_of(ki, b)

                @pl.when(blk < nfull)
                def _():
                    i_copy(blk, b, CH).wait()

                    @pl.when(ki >= NBUF)
                    def _():
                        o_copy(blk_of(ki - NBUF, b), b, CH).wait()

                    transpose_block(b, GROUPS)
                    o_copy(blk, b, CH).start()
                    nxt = blk_of(ki + NBUF, b)

                    @pl.when(nxt < nfull)
                    def _():
                        i_copy(nxt, b, CH).start()
            return 0

        lax.fori_loop(0, (nper + NBUF - 1) // NBUF, steady, 0)
        for b in range(NBUF):
            o_copy(0, b, CH).wait()

        if rem:
            @pl.when(wid == 0)
            def _():
                pltpu.sync_copy(tail_hbm, ob_v.at[pl.ds(0, rem)])
                pltpu.sync_copy(ob_v.at[pl.ds(0, rem)],
                                out_hbm.at[pl.ds(nfull * CH, rem)])

    return kern


@functools.lru_cache(maxsize=None)
def _build(nch, n_batch, vpad):
    mesh = plsc.VectorSubcoreMesh(core_axis_name="c", subcore_axis_name="s")

    @functools.partial(
        pl.kernel,
        mesh=mesh,
        compiler_params=pltpu.CompilerParams(needs_layout_passes=False),
        out_type=jax.ShapeDtypeStruct((nch, HDIM, n_batch), jnp.float32),
        scratch_types=[
            pltpu.VMEM((nch, CH), jnp.int32),
            pltpu.VMEM((NBUF * CH, 2 * HDIM), jnp.float32),
            pltpu.VMEM((NBUF * HDIM, CH), jnp.float32),
            pltpu.VMEM((HDIM * LANES,), jnp.float32),
            pltpu.VMEM((HDIM * LANES,), jnp.float32),
            pltpu.SemaphoreType.DMA((NBUF,)),
            pltpu.SemaphoreType.DMA((NBUF,)),
        ],
    )
    def kern(idx_hbm, table_hbm, gexp_hbm, bexp_hbm, out_hbm,
             idx_v, rows_v, obuf_v, gexp_v, bexp_v, gsem, osem):
        wid = lax.axis_index("s") * NC + lax.axis_index("c")
        pltpu.sync_copy(idx_hbm.at[wid], idx_v)
        pltpu.sync_copy(gexp_hbm, gexp_v)
        pltpu.sync_copy(bexp_hbm, bexp_v)
        rid0 = lax.iota(jnp.int32, LANES)

        def g_copy(ci, b):
            return pltpu.make_async_copy(
                table_hbm.at[idx_v.at[ci]],
                rows_v.at[pl.ds(b * CH, CH)], gsem.at[b])

        def o_copy(ci, b):
            return pltpu.make_async_copy(
                obuf_v.at[pl.ds(b * HDIM, HDIM)],
                out_hbm.at[ci, :, pl.ds(wid * CH, CH)], osem.at[b])

        def compute(ci, b):
            rowbase = b * CH
            obase = b * HDIM
            rids = [rid0 + (g * LANES + rowbase) for g in range(GROUPS)]
            tids = [rid0 + g * LANES for g in range(GROUPS)]

            zero = jnp.zeros((LANES,), jnp.float32)
            carry0 = tuple([zero] * (2 * GROUPS))

            def stats_body(d, carry):
                accs = list(carry)
                c63 = (rid0 + d) & (HDIM - 1)
                for g in range(GROUPS):
                    v = plsc.load_gather(rows_v, [rids[g], c63])
                    accs[g] = accs[g] + v
                    accs[GROUPS + g] = accs[GROUPS + g] + v * v
                return tuple(accs)

            res = plsc.parallel_loop(0, HDIM, carry=carry0)(stats_body)
            means, rstds = [], []
            for g in range(GROUPS):
                mean = res[g] * (1.0 / HDIM)
                var = res[GROUPS + g] * (1.0 / HDIM) - mean * mean
                means.append(mean)
                rstds.append(_rsqrt(var + EPS))

            @plsc.parallel_loop(0, HDIM, unroll=4)
            def phase_b(d):
                gd = gexp_v[pl.ds(d * LANES, LANES)]
                bd = bexp_v[pl.ds(d * LANES, LANES)]
                c63 = (rid0 + d) & (HDIM - 1)
                crow = c63 + obase
                for g in range(GROUPS):
                    v = plsc.load_gather(rows_v, [rids[g], c63])
                    o = (v - means[g]) * rstds[g] * gd + bd
                    plsc.store_scatter(obuf_v, [crow, tids[g]], o)

        for b in range(NBUF):
            g_copy(b, b).start()

        def steady(i, _):
            i0 = i * NBUF
            for b in range(NBUF):
                ci = i0 + b
                g_copy(ci, b).wait()

                @pl.when(ci >= NBUF)
                def _():
                    o_copy(ci - NBUF, b).wait()

                compute(ci, b)
                o_copy(ci, b).start()

                @pl.when(ci + NBUF < nch)
                def _():
                    g_copy(ci + NBUF, b).start()
            return 0

        lax.fori_loop(0, nch // NBUF, steady, 0)
        for b in range(NBUF):
            o_copy(nch - NBUF + b, b).wait()

    return kern


def kernel(input, table, gamma, beta):
    B, L = input.shape
    V, H = table.shape
    idx4 = input.reshape(NW, CH, L).transpose(0, 2, 1).astype(jnp.int32)
    nfull = V // CH
    tail = jnp.pad(table[nfull * CH:], ((0, 0), (0, H)))
    table2 = _build_fmt(V)(table.T, tail)
    diag = (jnp.arange(H)[:, None] + jnp.arange(LANES)[None, :]) % H
    gexp = gamma.astype(jnp.float32)[diag].reshape(H * LANES)
    bexp = beta.astype(jnp.float32)[diag].reshape(H * LANES)
    o2 = _build(L, B, table2.shape[0])(idx4, table2, gexp, bexp)
    return jnp.transpose(o2, (2, 0, 1))
